# Initial kernel scaffold; baseline (speedup 1.0000x reference)
import jax
import jax.numpy as jnp
from jax.experimental import pallas as pl


def kernel(x, edge_index, Wl0, Wr0, b0, g0, be0, Wl1, Wr1, b1, g1, be1, Wl2, Wr2, b2, g2, be2, Wc1, bc1, Wc2, bc2):
    raise NotImplementedError("write your pallas kernel here")


# speedup vs baseline: 3.7268x; 3.7268x over previous
import functools

import jax
import jax.numpy as jnp
from jax import lax
from jax.experimental import pallas as pl
from jax.experimental.pallas import tpu as pltpu
from jax.experimental.pallas import tpu_sc as plsc

_N = 10000
_D = 128
_CHUNK = 128
_NC = 2
_NS = 16
_NW = _NC * _NS
_ACC_ROWS = 10240
_RPT = _ACC_ROWS // _NS
_NCH_FULL = _N // _CHUNK
_TAIL = _N - _NCH_FULL * _CHUNK


def _build_msum(cpt):
  mesh = plsc.VectorSubcoreMesh(core_axis_name="c", subcore_axis_name="s")
  scratch = (
      pltpu.VMEM((_CHUNK,), jnp.int32),
      pltpu.VMEM((_CHUNK,), jnp.int32),
      pltpu.VMEM((_CHUNK, _D), jnp.float32),
      pltpu.VMEM_SHARED((_ACC_ROWS, _D), jnp.float32),
      pltpu.SemaphoreType.DMA,
  )

  @functools.partial(
      pl.kernel, mesh=mesh, scratch_types=scratch,
      out_type=(jax.ShapeDtypeStruct((_NC, _N, _D), jnp.float32),))
  def k(h_hbm, src_hbm, dst_hbm, zrow_hbm, msum_hbm,
        sidx, didx, rows, acc, sem):
    c = lax.axis_index("c")
    s = lax.axis_index("s")
    wid = s * _NC + c

    pltpu.sync_copy(zrow_hbm, rows)

    def zero_body(j, carry):
      r = s * _RPT + j * _CHUNK
      pltpu.sync_copy(rows, acc.at[pl.ds(r, _CHUNK)])
      return carry
    lax.fori_loop(0, _RPT // _CHUNK, zero_body, 0)

    plsc.subcore_barrier()

    def edge_body(j, carry):
      base = (wid * cpt + j) * _CHUNK
      pltpu.sync_copy(src_hbm.at[pl.ds(base, _CHUNK)], sidx)
      pltpu.sync_copy(dst_hbm.at[pl.ds(base, _CHUNK)], didx)
      pltpu.async_copy(h_hbm.at[sidx], rows, sem).wait()
      pltpu.sync_copy(rows, acc.at[didx], add=True)
      return carry
    lax.fori_loop(0, cpt, edge_body, 0)

    plsc.subcore_barrier()

    for k_ in range((_NCH_FULL + _NS) // _NS):
      idx = s + k_ * _NS

      @pl.when(idx < _NCH_FULL)
      def _full_chunk():
        r = idx * _CHUNK
        pltpu.sync_copy(acc.at[pl.ds(r, _CHUNK)], rows)
        pltpu.sync_copy(rows, msum_hbm.at[c, pl.ds(r, _CHUNK)])

      @pl.when(idx == _NCH_FULL)
      def _tail_chunk():
        r = _NCH_FULL * _CHUNK
        pltpu.sync_copy(acc.at[pl.ds(r, _TAIL)], rows.at[pl.ds(0, _TAIL)])
        pltpu.sync_copy(rows.at[pl.ds(0, _TAIL)],
                        msum_hbm.at[c, pl.ds(r, _TAIL)])

  return k


def _build_count(cpt):
  mesh = plsc.VectorSubcoreMesh(core_axis_name="c", subcore_axis_name="s")
  scratch = (
      pltpu.VMEM((_CHUNK,), jnp.int32),
      pltpu.VMEM((_CHUNK, _D), jnp.float32),
      pltpu.VMEM((_CHUNK, _D), jnp.float32),
      pltpu.VMEM_SHARED((_ACC_ROWS, _D), jnp.float32),
  )

  @functools.partial(
      pl.kernel, mesh=mesh, scratch_types=scratch,
      out_type=(jax.ShapeDtypeStruct((_NC, _N, _D), jnp.float32),))
  def k(dst_hbm, ones_hbm, zcnt_hbm, cnt_hbm, didx, ones, cbuf, cacc):
    c = lax.axis_index("c")
    s = lax.axis_index("s")
    wid = s * _NC + c

    pltpu.sync_copy(ones_hbm, ones)
    pltpu.sync_copy(zcnt_hbm, cbuf)

    def zero_body(j, carry):
      r = s * _RPT + j * _CHUNK
      pltpu.sync_copy(cbuf, cacc.at[pl.ds(r, _CHUNK)])
      return carry
    lax.fori_loop(0, _RPT // _CHUNK, zero_body, 0)

    plsc.subcore_barrier()

    def edge_body(j, carry):
      base = (wid * cpt + j) * _CHUNK
      pltpu.sync_copy(dst_hbm.at[pl.ds(base, _CHUNK)], didx)
      pltpu.sync_copy(ones, cacc.at[didx], add=True)
      return carry
    lax.fori_loop(0, cpt, edge_body, 0)

    plsc.subcore_barrier()

    for k_ in range((_NCH_FULL + _NS) // _NS):
      idx = s + k_ * _NS

      @pl.when(idx < _NCH_FULL)
      def _full_chunk():
        r = idx * _CHUNK
        pltpu.sync_copy(cacc.at[pl.ds(r, _CHUNK)], cbuf)
        pltpu.sync_copy(cbuf, cnt_hbm.at[c, pl.ds(r, _CHUNK)])

      @pl.when(idx == _NCH_FULL)
      def _tail_chunk():
        r = _NCH_FULL * _CHUNK
        pltpu.sync_copy(cacc.at[pl.ds(r, _TAIL)], cbuf.at[pl.ds(0, _TAIL)])
        pltpu.sync_copy(cbuf.at[pl.ds(0, _TAIL)],
                        cnt_hbm.at[c, pl.ds(r, _TAIL)])

  return k


def _combine(p, cnt, h, Wl, Wr, b, g, be):
  def body(p_ref, cnt_ref, h_ref, wl_ref, wr_ref, b_ref, g_ref, be_ref, o_ref):
    msum = p_ref[0, :, :] + p_ref[1, :, :]
    n_edges = cnt_ref[0, :, 0:1] + cnt_ref[1, :, 0:1]
    mean = msum / jnp.maximum(n_edges, 1.0)
    t = (jnp.dot(mean, wl_ref[...], preferred_element_type=jnp.float32)
         + jnp.dot(h_ref[...], wr_ref[...], preferred_element_type=jnp.float32)
         + b_ref[...])
    mu = jnp.mean(t, axis=0, keepdims=True)
    var = jnp.mean(jnp.square(t - mu), axis=0, keepdims=True)
    o_ref[...] = jnp.maximum(
        (t - mu) * lax.rsqrt(var + 1e-5) * g_ref[...] + be_ref[...], 0.0)

  return pl.pallas_call(
      body, out_shape=jax.ShapeDtypeStruct((_N, _D), jnp.float32),
  )(p, cnt, h, Wl, Wr, b.reshape(1, _D), g.reshape(1, _D), be.reshape(1, _D))


def _head(h1, h2, h3, w1a, w1b, w1c, bc1p, w2p, bc2p):
  def body(h1_ref, h2_ref, h3_ref, a_ref, b_ref, c_ref, bc1_ref, w2_ref,
           bc2_ref, o_ref):
    z = (jnp.dot(h1_ref[...], a_ref[...], preferred_element_type=jnp.float32)
         + jnp.dot(h2_ref[...], b_ref[...], preferred_element_type=jnp.float32)
         + jnp.dot(h3_ref[...], c_ref[...], preferred_element_type=jnp.float32)
         + bc1_ref[...])
    z = jnp.maximum(z, 0.0)
    o_ref[...] = (jnp.dot(z, w2_ref[...], preferred_element_type=jnp.float32)
                  + bc2_ref[...])

  return pl.pallas_call(
      body, out_shape=jax.ShapeDtypeStruct((_N, 128), jnp.float32),
  )(h1, h2, h3, w1a, w1b, w1c, bc1p, w2p, bc2p)


def kernel(x, edge_index, Wl0, Wr0, b0, g0, be0, Wl1, Wr1, b1, g1, be1,
           Wl2, Wr2, b2, g2, be2, Wc1, bc1, Wc2, bc2):
  E = edge_index.shape[1]
  cpt = -(-E // (_NW * _CHUNK))
  E2 = cpt * _NW * _CHUNK
  src = edge_index[0]
  dst = edge_index[1]
  if E2 != E:
    pad = E2 - E
    src = jnp.concatenate([src, jnp.zeros((pad,), jnp.int32)])
    dst = jnp.concatenate([dst, jnp.full((pad,), _N, jnp.int32)])

  zrow = jnp.zeros((_CHUNK, _D), jnp.float32)
  ones128 = jnp.ones((_CHUNK, _D), jnp.float32)

  msum_k = _build_msum(cpt)
  count_k = _build_count(cpt)

  (cnt,) = count_k(dst, ones128, zrow)
  (p0,) = msum_k(x, src, dst, zrow)
  h1 = _combine(p0, cnt, x, Wl0, Wr0, b0, g0, be0)
  (p1,) = msum_k(h1, src, dst, zrow)
  h2 = _combine(p1, cnt, h1, Wl1, Wr1, b1, g1, be1)
  (p2,) = msum_k(h2, src, dst, zrow)
  h3 = _combine(p2, cnt, h2, Wl2, Wr2, b2, g2, be2)

  hh = Wc1.shape[1]
  w1a = jnp.pad(Wc1[0:_D], ((0, 0), (0, 128 - hh)))
  w1b = jnp.pad(Wc1[_D:2 * _D], ((0, 0), (0, 128 - hh)))
  w1c = jnp.pad(Wc1[2 * _D:3 * _D], ((0, 0), (0, 128 - hh)))
  bc1p = jnp.pad(bc1.reshape(1, hh), ((0, 0), (0, 128 - hh)))
  w2p = jnp.pad(Wc2, ((0, 128 - hh), (0, 128 - Wc2.shape[1])))
  bc2p = jnp.pad(bc2.reshape(1, -1), ((0, 0), (0, 128 - Wc2.shape[1])))

  out = _head(h1, h2, h3, w1a, w1b, w1c, bc1p, w2p, bc2p)
  return out[:, :Wc2.shape[1]]
